# Initial kernel scaffold; baseline (speedup 1.0000x reference)
import jax
import jax.numpy as jnp
from jax.experimental import pallas as pl


def kernel(views, expert_keys, W1, b1, W2, b2, Wr, br):
    raise NotImplementedError("write your pallas kernel here")


# speedup vs baseline: 4.0494x; 4.0494x over previous
import functools

import jax
import jax.numpy as jnp
from jax.experimental import pallas as pl
from jax.experimental.pallas import tpu as pltpu

L_TOKENS = 2048
D_MODEL = 768
N_EXPERTS = 8
EPAD = 128


def _routing_kernel(h_ref, m_ref, bias_ref, w_ref):
    logits = jnp.dot(h_ref[...], m_ref[...], preferred_element_type=jnp.float32,
                     precision=jax.lax.Precision.HIGHEST)
    logits = logits + bias_ref[...]
    lane = jax.lax.broadcasted_iota(jnp.int32, logits.shape, 1)
    v1 = jnp.max(logits, axis=1, keepdims=True)
    e1 = jnp.min(jnp.where(logits == v1, lane, EPAD), axis=1, keepdims=True)
    l2 = jnp.where(lane == e1, -jnp.inf, logits)
    v2 = jnp.max(l2, axis=1, keepdims=True)
    e2 = jnp.min(jnp.where(l2 == v2, lane, EPAD), axis=1, keepdims=True)
    w1 = 1.0 / (1.0 + jnp.exp(v2 - v1))
    w2 = 1.0 - w1
    w_ref[...] = jnp.where(lane == e1, w1, 0.0) + jnp.where(lane == e2, w2, 0.0)


def _dense_moe_kernel(h_ref, w1_ref, b1_ref, w2_ref, b2_ref, gate_ref, out_ref,
                      *, tile: int):
    e = pl.program_id(0)
    t = pl.program_id(1)
    h = h_ref[...]
    hid = jnp.dot(h, w1_ref[0], preferred_element_type=jnp.float32)
    hid = hid + b1_ref[0]
    hid = hid * 0.5 * (1.0 + jax.lax.erf(hid * 0.7071067811865476))
    out = jnp.dot(hid, w2_ref[0], preferred_element_type=jnp.float32)
    out = out + b2_ref[0]
    lane = jax.lax.broadcasted_iota(jnp.int32, gate_ref.shape, 1)
    gate = jnp.sum(jnp.where(lane == e, gate_ref[...], 0.0), axis=1,
                   keepdims=True)
    val = gate * out
    sl = pl.ds(t * tile, tile)

    @pl.when(e == 0)
    def _init():
        out_ref[sl, :] = val

    @pl.when(e != 0)
    def _acc():
        out_ref[sl, :] += val


def _moe_one_view(h, expert_keys, W1, b1, W2, b2, Wr_v, br_v):
    L, D = h.shape
    E = N_EXPERTS
    m = 2.0 * expert_keys.T + Wr_v
    bias = br_v - jnp.sum(expert_keys * expert_keys, axis=1)
    m_pad = jnp.zeros((D, EPAD), jnp.float32).at[:, :E].set(m)
    bias_pad = jnp.full((1, EPAD), -jnp.inf, jnp.float32).at[0, :E].set(bias)

    gates = pl.pallas_call(
        _routing_kernel,
        out_shape=jax.ShapeDtypeStruct((L, EPAD), jnp.float32),
    )(h, m_pad, bias_pad)

    tile = 256
    T = L // tile
    grid = (E, T)
    out = pl.pallas_call(
        functools.partial(_dense_moe_kernel, tile=tile),
        grid=grid,
        in_specs=[
            pl.BlockSpec((tile, D), lambda e, t: (t, 0)),
            pl.BlockSpec((1, D, 4 * D), lambda e, t: (e, 0, 0)),
            pl.BlockSpec((1, 1, 4 * D), lambda e, t: (e, 0, 0)),
            pl.BlockSpec((1, 4 * D, D), lambda e, t: (e, 0, 0)),
            pl.BlockSpec((1, 1, D), lambda e, t: (e, 0, 0)),
            pl.BlockSpec((tile, EPAD), lambda e, t: (t, 0)),
        ],
        out_specs=pl.BlockSpec((L, D), lambda e, t: (0, 0)),
        out_shape=jax.ShapeDtypeStruct((L, D), jnp.float32),
        compiler_params=pltpu.CompilerParams(
            dimension_semantics=("arbitrary", "arbitrary"),
        ),
    )(h, W1, b1.reshape(E, 1, 4 * D), W2, b2.reshape(E, 1, D), gates)
    return out


def kernel(views, expert_keys, W1, b1, W2, b2, Wr, br):
    n_views, B, L, D = views.shape
    fused = jnp.zeros((B, L, D), views.dtype)
    for v in range(n_views):
        for b in range(B):
            out = _moe_one_view(views[v, b], expert_keys, W1, b1, W2, b2,
                                Wr[v], br[v])
            fused = fused.at[b].add(out)
    return fused


# speedup vs baseline: 4.1409x; 1.0226x over previous
import functools

import jax
import jax.numpy as jnp
from jax import lax
from jax.experimental import pallas as pl
from jax.experimental.pallas import tpu as pltpu
from jax.experimental.pallas import tpu_sc as plsc

L_TOKENS = 2048
D_MODEL = 768
N_EXPERTS = 8
EPAD = 128
TILE_M = 128
G_TILES = 40
P_ROWS = G_TILES * TILE_M
NW = 32
ROWS_PER_W = P_ROWS // NW
TOK_PER_W = L_TOKENS // NW
_SQRT_HALF = 0.7071067811865476


def _route_meta_kernel(m_ref, h_ref, bias_ref, p_ref, w_ref, cnt_ref):
    logits = lax.dot_general(
        m_ref[...], h_ref[...], (((0,), (1,)), ((), ())),
        preferred_element_type=jnp.float32, precision=lax.Precision.HIGHEST)
    logits = logits + bias_ref[...]
    erow = lax.broadcasted_iota(jnp.int32, (EPAD, L_TOKENS), 0)
    v1 = jnp.max(logits, axis=0, keepdims=True)
    e1 = jnp.min(jnp.where(logits == v1, erow, EPAD), axis=0, keepdims=True)
    l2 = jnp.where(erow == e1, -jnp.inf, logits)
    v2 = jnp.max(l2, axis=0, keepdims=True)
    e2 = jnp.min(jnp.where(l2 == v2, erow, EPAD), axis=0, keepdims=True)
    w1 = 1.0 / (1.0 + jnp.exp(v2 - v1))
    member = jnp.logical_or(erow == e1, erow == e2).astype(jnp.float32)
    t_i = lax.broadcasted_iota(jnp.int32, (L_TOKENS, L_TOKENS), 0)
    t_j = lax.broadcasted_iota(jnp.int32, (L_TOKENS, L_TOKENS), 1)
    ut = (t_i < t_j).astype(jnp.float32)
    rank = lax.dot_general(member, ut, (((1,), (0,)), ((), ())),
                           preferred_element_type=jnp.float32)
    cnt = jnp.sum(member, axis=1, keepdims=True)
    ntiles = jnp.floor((cnt + (TILE_M - 1)) * (1.0 / TILE_M))
    e_i = lax.broadcasted_iota(jnp.int32, (EPAD, EPAD), 0)
    e_j = lax.broadcasted_iota(jnp.int32, (EPAD, EPAD), 1)
    ute = (e_i < e_j).astype(jnp.float32)
    poff = TILE_M * lax.dot_general(ute, ntiles, (((0,), (0,)), ((), ())),
                                    preferred_element_type=jnp.float32)
    base = rank + poff
    p1 = jnp.sum(jnp.where(erow == e1, base, 0.0), axis=0, keepdims=True)
    p2 = jnp.sum(jnp.where(erow == e2, base, 0.0), axis=0, keepdims=True)
    p_ref[0:1, :] = p1.astype(jnp.int32)
    p_ref[1:2, :] = p2.astype(jnp.int32)
    w_ref[0:1, :] = w1
    w_ref[1:2, :] = 1.0 - w1
    cnt_ref[...] = cnt.astype(jnp.int32)


def _sc_dispatch_body(p_hbm, w_hbm, h_hbm, x_out, w_out,
                      p_v, w_v, tok_v, wloc_v, rows_v, sem):
    wid = lax.axis_index("s") * 2 + lax.axis_index("c")
    base = wid * ROWS_PER_W
    pltpu.sync_copy(p_hbm, p_v)
    pltpu.sync_copy(w_hbm, w_v)
    for kk in range(ROWS_PER_W // 16):
        tok_v[pl.ds(kk * 16, 16)] = jnp.zeros((16,), jnp.int32)
        wloc_v[pl.ds(kk * 16, 16)] = jnp.zeros((16,), jnp.float32)
    iot = lax.broadcasted_iota(jnp.int32, (16,), 0)

    def body(i2, carry):
        tvec = i2 * 16 + iot
        for j in range(2):
            vp = p_v[j, pl.ds(i2 * 16, 16)]
            vw = w_v[j, pl.ds(i2 * 16, 16)]
            msk = jnp.logical_and(vp >= base, vp < base + ROWS_PER_W)
            idx = jnp.where(msk, vp - base, 0)
            plsc.store_scatter(tok_v, [idx], tvec, mask=msk)
            plsc.store_scatter(wloc_v, [idx], vw, mask=msk)
        return carry

    lax.fori_loop(0, L_TOKENS // 16, body, 0)
    pltpu.sync_copy(wloc_v, w_out.at[pl.ds(base, ROWS_PER_W)])
    half = ROWS_PER_W // 2
    for c in range(2):
        pltpu.async_copy(h_hbm.at[tok_v.at[pl.ds(c * half, half)]],
                         rows_v, sem).wait()
        pltpu.sync_copy(rows_v, x_out.at[pl.ds(base + c * half, half), :])


def _gmm_kernel(te_ref, x_ref, w1_ref, b1_ref, w2_ref, b2_ref, wr_ref, y_ref):
    del te_ref
    hid = jnp.dot(x_ref[...], w1_ref[0], preferred_element_type=jnp.float32)
    hid = hid + b1_ref[0]
    hid = hid * 0.5 * (1.0 + lax.erf(hid * _SQRT_HALF))
    out = jnp.dot(hid, w2_ref[0], preferred_element_type=jnp.float32)
    y_ref[...] = (out + b2_ref[0]) * wr_ref[...]


def _sc_combine_body(p_hbm, y_hbm, out_hbm, p0_v, p1_v, buf0, buf1, sem):
    wid = lax.axis_index("s") * 2 + lax.axis_index("c")
    tbase = wid * TOK_PER_W
    pltpu.sync_copy(p_hbm.at[0, pl.ds(tbase, TOK_PER_W)], p0_v)
    pltpu.sync_copy(p_hbm.at[1, pl.ds(tbase, TOK_PER_W)], p1_v)
    half = TOK_PER_W // 2
    for c in range(2):
        pltpu.async_copy(y_hbm.at[p0_v.at[pl.ds(c * half, half)]],
                         buf0, sem).wait()
        pltpu.async_copy(y_hbm.at[p1_v.at[pl.ds(c * half, half)]],
                         buf1, sem).wait()

        def addbody(r, carry):
            for cc in range(D_MODEL // 16):
                sl = pl.ds(cc * 16, 16)
                buf0[r, sl] = buf0[r, sl] + buf1[r, sl]
            return carry

        lax.fori_loop(0, half, addbody, 0)
        pltpu.sync_copy(buf0, out_hbm.at[pl.ds(tbase + c * half, half), :])


@functools.lru_cache(maxsize=1)
def _sc_kernels():
    mesh = plsc.VectorSubcoreMesh(core_axis_name="c", subcore_axis_name="s")
    dispatch = functools.partial(
        pl.kernel, mesh=mesh,
        out_type=[jax.ShapeDtypeStruct((P_ROWS, D_MODEL), jnp.float32),
                  jax.ShapeDtypeStruct((P_ROWS,), jnp.float32)],
        scratch_types=[pltpu.VMEM((2, L_TOKENS), jnp.int32),
                       pltpu.VMEM((2, L_TOKENS), jnp.float32),
                       pltpu.VMEM((ROWS_PER_W,), jnp.int32),
                       pltpu.VMEM((ROWS_PER_W,), jnp.float32),
                       pltpu.VMEM((ROWS_PER_W // 2, D_MODEL), jnp.float32),
                       pltpu.SemaphoreType.DMA],
        compiler_params=pltpu.CompilerParams(needs_layout_passes=False),
    )(_sc_dispatch_body)
    combine = functools.partial(
        pl.kernel, mesh=mesh,
        out_type=jax.ShapeDtypeStruct((L_TOKENS, D_MODEL), jnp.float32),
        scratch_types=[pltpu.VMEM((TOK_PER_W,), jnp.int32),
                       pltpu.VMEM((TOK_PER_W,), jnp.int32),
                       pltpu.VMEM((TOK_PER_W // 2, D_MODEL), jnp.float32),
                       pltpu.VMEM((TOK_PER_W // 2, D_MODEL), jnp.float32),
                       pltpu.SemaphoreType.DMA],
        compiler_params=pltpu.CompilerParams(needs_layout_passes=False),
    )(_sc_combine_body)
    return dispatch, combine


def _route_meta(h, expert_keys, Wr_v, br_v):
    m = 2.0 * expert_keys.T + Wr_v
    bias = br_v - jnp.sum(expert_keys * expert_keys, axis=1)
    m_pad = jnp.zeros((D_MODEL, EPAD), jnp.float32).at[:, :N_EXPERTS].set(m)
    bias_pad = jnp.full((EPAD, 1), -jnp.inf,
                        jnp.float32).at[:N_EXPERTS, 0].set(bias)
    return pl.pallas_call(
        _route_meta_kernel,
        out_shape=[jax.ShapeDtypeStruct((2, L_TOKENS), jnp.int32),
                   jax.ShapeDtypeStruct((2, L_TOKENS), jnp.float32),
                   jax.ShapeDtypeStruct((EPAD, 1), jnp.int32)],
    )(m_pad, h, bias_pad)


def _tile_expert_map(cnt):
    cnt8 = cnt[:N_EXPERTS, 0]
    nt = (cnt8 + TILE_M - 1) // TILE_M
    cum = jnp.cumsum(nt)
    gidx = jnp.arange(G_TILES, dtype=jnp.int32)
    te = jnp.sum((cum[None, :] <= gidx[:, None]).astype(jnp.int32), axis=1)
    return jnp.minimum(te, N_EXPERTS - 1).astype(jnp.int32)


def _gmm(te, x_sorted, W1, b1, W2, b2, row_w):
    E = N_EXPERTS
    D = D_MODEL
    grid_spec = pltpu.PrefetchScalarGridSpec(
        num_scalar_prefetch=1,
        grid=(G_TILES,),
        in_specs=[
            pl.BlockSpec((TILE_M, D), lambda g, te: (g, 0)),
            pl.BlockSpec((1, D, 4 * D), lambda g, te: (te[g], 0, 0)),
            pl.BlockSpec((1, 1, 4 * D), lambda g, te: (te[g], 0, 0)),
            pl.BlockSpec((1, 4 * D, D), lambda g, te: (te[g], 0, 0)),
            pl.BlockSpec((1, 1, D), lambda g, te: (te[g], 0, 0)),
            pl.BlockSpec((TILE_M, 1), lambda g, te: (g, 0)),
        ],
        out_specs=pl.BlockSpec((TILE_M, D), lambda g, te: (g, 0)),
    )
    return pl.pallas_call(
        _gmm_kernel,
        grid_spec=grid_spec,
        out_shape=jax.ShapeDtypeStruct((P_ROWS, D), jnp.float32),
        compiler_params=pltpu.CompilerParams(
            dimension_semantics=("arbitrary",),
        ),
    )(te, x_sorted, W1, b1.reshape(E, 1, 4 * D), W2, b2.reshape(E, 1, D),
      row_w.reshape(P_ROWS, 1))


def _moe_one_view(h, expert_keys, W1, b1, W2, b2, Wr_v, br_v):
    asg_p, asg_w, cnt = _route_meta(h, expert_keys, Wr_v, br_v)
    te = _tile_expert_map(cnt)
    sc_dispatch, sc_combine = _sc_kernels()
    x_sorted, row_w = sc_dispatch(asg_p, asg_w, h)
    y = _gmm(te, x_sorted, W1, b1, W2, b2, row_w)
    return sc_combine(asg_p, y)


def kernel(views, expert_keys, W1, b1, W2, b2, Wr, br):
    n_views, B, L, D = views.shape
    fused = jnp.zeros((B, L, D), views.dtype)
    for v in range(n_views):
        for b in range(B):
            out = _moe_one_view(views[v, b], expert_keys, W1, b1, W2, b2,
                                Wr[v], br[v])
            fused = fused.at[b].add(out)
    return fused
